# Initial kernel scaffold; baseline (speedup 1.0000x reference)
import jax
import jax.numpy as jnp
from jax.experimental import pallas as pl


def kernel(h, edge_index, U0, ptr, W_spec, lin_W, lin_b, ln_g, ln_b):
    raise NotImplementedError("write your pallas kernel here")


# speedup vs baseline: 5.7425x; 5.7425x over previous
import functools
import math

import jax
import jax.numpy as jnp
from jax import lax
from jax.experimental import pallas as pl
from jax.experimental.pallas import tpu as pltpu
from jax.experimental.pallas import tpu_sc as plsc

N = 10000
E = 320000
C = 128
M = 64
CH = C // 2
NS = 16
NP = 10240
STRIPE = NP // NS
EPT = E // NS
K = 80
NCHUNK = EPT // K
DEGW = 16


def _sc_agg_build():
    mesh = plsc.VectorSubcoreMesh(core_axis_name="c", subcore_axis_name="s")

    @functools.partial(
        pl.kernel,
        mesh=mesh,
        compiler_params=pltpu.CompilerParams(use_tc_tiling_on_sc=False),
        out_type=[
            jax.ShapeDtypeStruct((2, NP, CH), jnp.float32),
            jax.ShapeDtypeStruct((2, NP, DEGW), jnp.float32),
        ],
        scratch_types=[
            pltpu.VMEM_SHARED((NP, CH), jnp.float32),
            pltpu.VMEM_SHARED((NP, DEGW), jnp.float32),
            pltpu.VMEM((NCHUNK, K), jnp.int32),
            pltpu.VMEM((NCHUNK, K), jnp.int32),
            pltpu.VMEM((K, CH), jnp.float32),
            pltpu.VMEM((K, DEGW), jnp.float32),
        ],
    )
    def sc_agg(h2, srcr, dstr, ones_h, zeros_h, zeros_d, agg_out, deg_out,
               sh_agg, sh_deg, src_slab, dst_slab, rows, ones_v):
        c = lax.axis_index("c")
        s = lax.axis_index("s")
        base = s * STRIPE
        pltpu.sync_copy(srcr.at[s], src_slab)
        pltpu.sync_copy(dstr.at[s], dst_slab)
        pltpu.sync_copy(ones_h, ones_v)
        pltpu.sync_copy(zeros_h, sh_agg.at[pl.ds(base, STRIPE)])
        pltpu.sync_copy(zeros_d, sh_deg.at[pl.ds(base, STRIPE)])
        plsc.subcore_barrier()

        half = NCHUNK // 2

        def body(j, carry):
            pltpu.sync_copy(h2.at[c].at[src_slab.at[j]], rows)
            pltpu.sync_copy(rows, sh_agg.at[dst_slab.at[j]], add=True)
            do_deg = jnp.where(c == 0, j < half, j >= half)

            @pl.when(do_deg)
            def _():
                pltpu.sync_copy(ones_v, sh_deg.at[dst_slab.at[j]], add=True)

            return carry

        lax.fori_loop(0, NCHUNK, body, 0)
        plsc.subcore_barrier()
        pltpu.sync_copy(sh_agg.at[pl.ds(base, STRIPE)],
                        agg_out.at[c, pl.ds(base, STRIPE)])
        pltpu.sync_copy(sh_deg.at[pl.ds(base, STRIPE)],
                        deg_out.at[c, pl.ds(base, STRIPE)])

    return sc_agg


_sc_agg = _sc_agg_build()


def _tc_spectral(u_ref, h_ref, w_ref, o_ref, acc_ref):
    i = pl.program_id(0)

    @pl.when(i == 0)
    def _():
        acc_ref[...] = jnp.zeros_like(acc_ref)

    acc_ref[...] += lax.dot_general(
        u_ref[...], h_ref[...], (((0,), (0,)), ((), ())),
        preferred_element_type=jnp.float32)

    @pl.when(i == pl.num_programs(0) - 1)
    def _():
        hh = acc_ref[...]
        o_ref[...] = jnp.sum(hh[:, :, None] * w_ref[...], axis=1)


def _tc_epilogue(h_ref, u_ref, oh_ref, agg_ref, deg_ref, w_ref, b_ref,
                 g_ref, bb_ref, out_ref):
    spec = lax.dot_general(u_ref[...], oh_ref[...], (((1,), (0,)), ((), ())),
                           preferred_element_type=jnp.float32)
    deg = deg_ref[0, :, 0:1] + deg_ref[1, :, 0:1]
    inv = 1.0 / jnp.maximum(deg, 1.0)
    lo = agg_ref[0] * inv
    hi = agg_ref[1] * inv
    local = (
        lax.dot_general(lo, w_ref[:, :CH], (((1,), (1,)), ((), ())),
                        preferred_element_type=jnp.float32)
        + lax.dot_general(hi, w_ref[:, CH:], (((1,), (1,)), ((), ())),
                          preferred_element_type=jnp.float32)
        + b_ref[...]
    )
    x = h_ref[...] + spec + local
    mu = jnp.mean(x, axis=-1, keepdims=True)
    xc = x - mu
    var = jnp.mean(xc * xc, axis=-1, keepdims=True)
    xn = xc * lax.rsqrt(var + 1e-5) * g_ref[...] + bb_ref[...]
    out_ref[...] = 0.5 * xn * (1.0 + lax.erf(xn * (1.0 / math.sqrt(2.0))))


def kernel(h, edge_index, U0, ptr, W_spec, lin_W, lin_b, ln_g, ln_b):
    del ptr
    src = edge_index[0].reshape(NS, NCHUNK, K)
    dst = edge_index[1].reshape(NS, NCHUNK, K)
    hp = jnp.pad(h, ((0, NP - N), (0, 0)))
    h2 = jnp.stack([hp[:, :CH], hp[:, CH:]])
    ones_h = jnp.ones((K, DEGW), jnp.float32)
    zeros_h = jnp.zeros((STRIPE, CH), jnp.float32)
    zeros_d = jnp.zeros((STRIPE, DEGW), jnp.float32)

    agg2, deg2 = _sc_agg(h2, src, dst, ones_h, zeros_h, zeros_d)

    RB = 1000
    nb = N // RB
    out_hat = pl.pallas_call(
        _tc_spectral,
        grid=(nb,),
        in_specs=[
            pl.BlockSpec((RB, M), lambda i: (i, 0)),
            pl.BlockSpec((RB, C), lambda i: (i, 0)),
            pl.BlockSpec((M, C, C), lambda i: (0, 0, 0)),
        ],
        out_specs=pl.BlockSpec((M, C), lambda i: (0, 0)),
        out_shape=jax.ShapeDtypeStruct((M, C), jnp.float32),
        scratch_shapes=[pltpu.VMEM((M, C), jnp.float32)],
    )(U0, h, W_spec)

    out = pl.pallas_call(
        _tc_epilogue,
        grid=(nb,),
        in_specs=[
            pl.BlockSpec((RB, C), lambda i: (i, 0)),
            pl.BlockSpec((RB, M), lambda i: (i, 0)),
            pl.BlockSpec((M, C), lambda i: (0, 0)),
            pl.BlockSpec((2, RB, CH), lambda i: (0, i, 0)),
            pl.BlockSpec((2, RB, DEGW), lambda i: (0, i, 0)),
            pl.BlockSpec((C, C), lambda i: (0, 0)),
            pl.BlockSpec((1, C), lambda i: (0, 0)),
            pl.BlockSpec((1, C), lambda i: (0, 0)),
            pl.BlockSpec((1, C), lambda i: (0, 0)),
        ],
        out_specs=pl.BlockSpec((RB, C), lambda i: (i, 0)),
        out_shape=jax.ShapeDtypeStruct((N, C), jnp.float32),
    )(h, U0, out_hat, agg2, deg2, lin_W, lin_b.reshape(1, C),
      ln_g.reshape(1, C), ln_b.reshape(1, C))
    return out
